# baseline (speedup 1.0000x reference)
import functools
import jax
import jax.numpy as jnp
import numpy as np
from jax.experimental import pallas as pl
from jax.experimental.pallas import tpu as pltpu

HIDDEN = 2048; KNUM = 512; KDIM = 128; VDIM = 64; KNN = 32; HEAD = 2; RANK = 2; MHEAD = 2; VET = 4
KEY_NUM = 1024
VALUE_NUM = KNUM * KNUM
TOK_BLK = 256


def _keyln_body(k_ref, kn_w_ref, kn_b_ref, out_ref):
    k = k_ref[...]
    m = k.mean(-1, keepdims=True)
    v = ((k - m) ** 2).mean(-1, keepdims=True)
    out_ref[...] = (k - m) / jnp.sqrt(v + 1e-5) * kn_w_ref[...] + kn_b_ref[...]


def _score_body(x_ref, wq_ref, krow_ref, kcol_ref, qn_w_ref, qn_b_ref,
                srow_ref, scol_ref):
    x = x_ref[...]
    wq = wq_ref[...]
    q = jax.lax.dot_general(x, wq, (((1,), (1,)), ((), ())),
                            preferred_element_type=jnp.float32)
    qw = qn_w_ref[...]
    qb = qn_b_ref[...]

    def ln(h):
        m = h.mean(-1, keepdims=True)
        v = ((h - m) ** 2).mean(-1, keepdims=True)
        return (h - m) / jnp.sqrt(v + 1e-5) * qw + qb

    q1 = ln(q[:, :KDIM])
    q2 = ln(q[:, KDIM:])
    srow_ref[...] = jax.lax.dot_general(
        q1, krow_ref[...], (((1,), (1,)), ((), ())),
        preferred_element_type=jnp.float32)
    scol_ref[...] = jax.lax.dot_general(
        q2, kcol_ref[...], (((1,), (1,)), ((), ())),
        preferred_element_type=jnp.float32)


def kernel(hidden_state, Wq, keys, values, Wv, qn_w, qn_b, kn_w, kn_b, tucker_cores, shuffle_index):
    b0, s0 = hidden_state.shape[0], hidden_state.shape[1]
    bs = b0 * s0
    x = hidden_state.reshape(bs, HIDDEN)

    kt = jnp.transpose(keys, (1, 4, 0, 2, 3)).reshape(2, RANK * HEAD * KEY_NUM, KDIM)

    k_ln = pl.pallas_call(
        _keyln_body,
        out_shape=jax.ShapeDtypeStruct(kt.shape, jnp.float32),
    )(kt, kn_w.reshape(1, 1, KDIM), kn_b.reshape(1, 1, KDIM))
    krow = k_ln[0]
    kcol = k_ln[1]

    nblk = bs // TOK_BLK
    srow, scol = pl.pallas_call(
        _score_body,
        grid=(nblk,),
        in_specs=[
            pl.BlockSpec((TOK_BLK, HIDDEN), lambda i: (i, 0)),
            pl.BlockSpec((2 * KDIM, HIDDEN), lambda i: (0, 0)),
            pl.BlockSpec(krow.shape, lambda i: (0, 0)),
            pl.BlockSpec(kcol.shape, lambda i: (0, 0)),
            pl.BlockSpec((1, KDIM), lambda i: (0, 0)),
            pl.BlockSpec((1, KDIM), lambda i: (0, 0)),
        ],
        out_specs=[
            pl.BlockSpec((TOK_BLK, RANK * HEAD * KEY_NUM), lambda i: (i, 0)),
            pl.BlockSpec((TOK_BLK, RANK * HEAD * KEY_NUM), lambda i: (i, 0)),
        ],
        out_shape=[
            jax.ShapeDtypeStruct((bs, RANK * HEAD * KEY_NUM), jnp.float32),
            jax.ShapeDtypeStruct((bs, RANK * HEAD * KEY_NUM), jnp.float32),
        ],
    )(x, Wq, krow, kcol, qn_w.reshape(1, KDIM), qn_b.reshape(1, KDIM))

    S_row = jnp.transpose(srow.reshape(bs, RANK, HEAD, KEY_NUM), (0, 2, 3, 1))
    S_col = jnp.transpose(scol.reshape(bs, RANK, HEAD, KEY_NUM), (0, 2, 3, 1))

    core = tucker_cores.sum(0)
    U, _, Vh = jnp.linalg.svd(core)
    u = jax.lax.stop_gradient(U[..., 0])
    v = jax.lax.stop_gradient(Vh[..., 0, :])
    ar = jnp.einsum('bhkr,hr->bhk', S_row, u)
    ac = jnp.einsum('bhkr,hr->bhk', S_col, v)
    _, ri = jax.lax.top_k(ar, KNN)
    _, ci = jax.lax.top_k(ac, KNN)
    Sr = jnp.take_along_axis(S_row, ri[..., None], axis=2)
    Sc = jnp.take_along_axis(S_col, ci[..., None], axis=2)
    grid = jnp.einsum('bhir,hrs,bhjs->bhij', Sr, core, Sc)
    best_scores, fi = jax.lax.top_k(grid.reshape(bs, HEAD, KNN * KNN), KNN)
    rows = jnp.take_along_axis(ri, fi // KNN, axis=2)
    cols = jnp.take_along_axis(ci, fi % KNN, axis=2)
    best_indice = shuffle_index[rows * KEY_NUM + cols]
    w = jax.nn.softmax(best_scores, axis=-1)
    phys = best_indice // VET
    exp = best_indice % VET
    vals = values[phys] * w[..., None]
    oh = jax.nn.one_hot(exp, VET, dtype=vals.dtype)
    agg = jnp.einsum('bhkd,bhke->bed', vals, oh).reshape(bs, VET * VDIM)
    out = agg @ Wv.T
    return out.reshape(b0, s0, HIDDEN)


# speedup vs baseline: 5.7653x; 5.7653x over previous
import jax
import jax.numpy as jnp
from jax import lax
from jax.experimental import pallas as pl
from jax.experimental.pallas import tpu as pltpu

HIDDEN = 2048; KDIM = 128; VDIM = 64; KNN = 32; HEAD = 2; RANK = 2; VET = 4
KEY_NUM = 1024
HK = HEAD * KEY_NUM
TOK_BLK = 256
NEG = -3.0e38


def _bf(x):
    return x.astype(jnp.bfloat16).astype(jnp.float32)


def _ln_last(x, w, b, eps=1e-5):
    m = x.mean(-1, keepdims=True)
    v = ((x - m) ** 2).mean(-1, keepdims=True)
    return (x - m) / jnp.sqrt(v + eps) * w + b


def _prep_body(kt_ref, knw_ref, knb_ref, kr_ref, kc_ref):
    k_ln = _ln_last(kt_ref[...], knw_ref[...], knb_ref[...])
    kr_ref[...] = k_ln[0]
    kc_ref[...] = k_ln[1]


def _halves(a):
    return a[:, :KEY_NUM], a[:, KEY_NUM:]


def _hmax_bc(a):
    h0, h1 = _halves(a)
    T = a.shape[0]
    m0 = jnp.max(h0, axis=1, keepdims=True)
    m1 = jnp.max(h1, axis=1, keepdims=True)
    return jnp.concatenate([jnp.broadcast_to(m0, (T, KEY_NUM)),
                            jnp.broadcast_to(m1, (T, KEY_NUM))], axis=1)


def _hmin_bc_i32(a):
    h0, h1 = _halves(a)
    T = a.shape[0]
    m0 = jnp.min(h0, axis=1, keepdims=True)
    m1 = jnp.min(h1, axis=1, keepdims=True)
    return (jnp.concatenate([jnp.broadcast_to(m0, (T, KEY_NUM)),
                             jnp.broadcast_to(m1, (T, KEY_NUM))], axis=1),
            m0, m1)


def _hsum_pair(a):
    h0, h1 = _halves(a)
    return (jnp.sum(h0, axis=1, keepdims=True),
            jnp.sum(h1, axis=1, keepdims=True))


def _topk_loop(A, payloads, iota_l, want_scores=False):
    caps = [[[], []] for _ in payloads]
    idxs = [[], []]
    scores = [[], []]
    for _ in range(KNN):
        m_bc = _hmax_bc(A)
        if want_scores:
            scores[0].append(m_bc[:, :1])
            scores[1].append(m_bc[:, KEY_NUM:KEY_NUM + 1])
        eq = A == m_bc
        cand = jnp.where(eq, iota_l, 2 * KEY_NUM)
        sel_bc, s0, s1 = _hmin_bc_i32(cand)
        idxs[0].append(s0)
        idxs[1].append(s1)
        oh = iota_l == sel_bc
        A = jnp.where(oh, NEG, A)
        for ci, P in enumerate(payloads):
            z = jnp.where(oh, P, jnp.zeros_like(P))
            c0, c1 = _hsum_pair(z)
            caps[ci][0].append(c0)
            caps[ci][1].append(c1)
    cat = lambda lst: jnp.concatenate(lst, axis=1)
    out = []
    if want_scores:
        out.append((cat(scores[0]), cat(scores[1])))
    out.append((cat(idxs[0]), cat(idxs[1])))
    for ci in range(len(payloads)):
        out.append((cat(caps[ci][0]), cat(caps[ci][1])))
    return out


def _expand_i(a):
    T = a.shape[0]
    return jnp.concatenate(
        [jnp.broadcast_to(a[:, i:i + 1], (T, KNN)) for i in range(KNN)], axis=1)


def _tile_j(a):
    return jnp.concatenate([a] * KNN, axis=1)


def _select_body(x_ref, wq_ref, kr_ref, kc_ref, qnw_ref, qnb_ref,
                 u_ref, v_ref, c_ref, w_ref, idx_ref):
    T = TOK_BLK
    x = x_ref[...]
    q = lax.dot_general(x, wq_ref[...], (((1,), (1,)), ((), ())),
                        preferred_element_type=jnp.float32)
    qw = qnw_ref[...]
    qb = qnb_ref[...]
    q1 = _ln_last(q[:, :KDIM], qw, qb)
    q2 = _ln_last(q[:, KDIM:], qw, qb)

    def score(qq, kref):
        return lax.dot_general(qq, kref[...], (((1,), (1,)), ((), ())),
                               preferred_element_type=jnp.float32)

    SRf = score(q1, kr_ref)
    SCf = score(q2, kc_ref)
    SR0, SR1 = SRf[:, :HK], SRf[:, HK:]
    SC0, SC1 = SCf[:, :HK], SCf[:, HK:]

    iota_l = lax.broadcasted_iota(jnp.int32, (T, HK), 1) % KEY_NUM
    hmask = lax.broadcasted_iota(jnp.int32, (T, HK), 1) >= KEY_NUM

    def comb(P0, P1, s00, s10, s01, s11):
        c0 = _bf(jnp.where(hmask, s01, s00))
        c1 = _bf(jnp.where(hmask, s11, s10))
        return _bf(P0) * c0 + _bf(P1) * c1

    AR = comb(SR0, SR1, u_ref[0, 0], u_ref[0, 1], u_ref[1, 0], u_ref[1, 1])
    AC = comb(SC0, SC1, v_ref[0, 0], v_ref[0, 1], v_ref[1, 0], v_ref[1, 1])

    (ri0, ri1), (sr0h0, sr0h1), (sr1h0, sr1h1) = _topk_loop(
        AR, [SR0, SR1], iota_l)
    (ci0, ci1), (sc0h0, sc0h1), (sc1h0, sc1h1) = _topk_loop(
        AC, [SC0, SC1], iota_l)

    def grid_head(sr0, sr1, sc0, sc1, h):
        t0 = _bf(sr0) * _bf(c_ref[h, 0, 0]) + _bf(sr1) * _bf(c_ref[h, 1, 0])
        t1 = _bf(sr0) * _bf(c_ref[h, 0, 1]) + _bf(sr1) * _bf(c_ref[h, 1, 1])
        return (_expand_i(_bf(t0)) * _tile_j(_bf(sc0)) +
                _expand_i(_bf(t1)) * _tile_j(_bf(sc1)))

    G0 = grid_head(sr0h0, sr1h0, sc0h0, sc1h0, 0)
    G1 = grid_head(sr0h1, sr1h1, sc0h1, sc1h1, 1)
    G = jnp.concatenate([G0, G1], axis=1)
    RIo = jnp.concatenate([_expand_i(ri0), _expand_i(ri1)], axis=1)
    CIo = jnp.concatenate([_tile_j(ci0), _tile_j(ci1)], axis=1)
    RIof = RIo.astype(jnp.float32)
    CIof = CIo.astype(jnp.float32)

    (bs0, bs1), _, (r0, r1), (c0, c1) = _topk_loop(
        G, [RIof, CIof], iota_l, want_scores=True)

    def smax(s):
        mx = jnp.max(s, axis=1, keepdims=True)
        e = jnp.exp(s - mx)
        return e / jnp.sum(e, axis=1, keepdims=True)

    w_ref[...] = jnp.concatenate([smax(bs0), smax(bs1)], axis=1)
    gi0 = r0.astype(jnp.int32) * KEY_NUM + c0.astype(jnp.int32)
    gi1 = r1.astype(jnp.int32) * KEY_NUM + c1.astype(jnp.int32)
    idx_ref[...] = jnp.concatenate([gi0, gi1], axis=1)


def kernel(hidden_state, Wq, keys, values, Wv, qn_w, qn_b, kn_w, kn_b, tucker_cores, shuffle_index):
    b0, s0 = hidden_state.shape[0], hidden_state.shape[1]
    bs = b0 * s0
    x = hidden_state.reshape(bs, HIDDEN)

    core = tucker_cores.sum(0)
    U, _, Vh = jnp.linalg.svd(core)
    u = lax.stop_gradient(U[..., 0])
    v = lax.stop_gradient(Vh[..., 0, :])

    kt = jnp.transpose(keys, (1, 4, 0, 2, 3)).reshape(2, RANK * HK, KDIM)

    shp = jax.ShapeDtypeStruct((RANK * HK, KDIM), jnp.float32)
    kr, kc = pl.pallas_call(
        _prep_body,
        in_specs=[pl.BlockSpec((2, RANK * HK, KDIM), lambda: (0, 0, 0)),
                  pl.BlockSpec((1, 1, KDIM), lambda: (0, 0, 0)),
                  pl.BlockSpec((1, 1, KDIM), lambda: (0, 0, 0))],
        out_shape=[shp, shp],
    )(kt, kn_w.reshape(1, 1, KDIM), kn_b.reshape(1, 1, KDIM))

    nblk = bs // TOK_BLK
    full = lambda s: pl.BlockSpec(s, lambda i: (0, 0))
    smem = pl.BlockSpec(memory_space=pltpu.SMEM)
    w_all, idx_all = pl.pallas_call(
        _select_body,
        grid=(nblk,),
        in_specs=[
            pl.BlockSpec((TOK_BLK, HIDDEN), lambda i: (i, 0)),
            full((2 * KDIM, HIDDEN)),
            full((RANK * HK, KDIM)), full((RANK * HK, KDIM)),
            full((1, KDIM)), full((1, KDIM)),
            smem, smem, smem,
        ],
        out_specs=[
            pl.BlockSpec((TOK_BLK, HEAD * KNN), lambda i: (i, 0)),
            pl.BlockSpec((TOK_BLK, HEAD * KNN), lambda i: (i, 0)),
        ],
        out_shape=[
            jax.ShapeDtypeStruct((bs, HEAD * KNN), jnp.float32),
            jax.ShapeDtypeStruct((bs, HEAD * KNN), jnp.int32),
        ],
    )(x, Wq, kr, kc, qn_w.reshape(1, KDIM), qn_b.reshape(1, KDIM),
      u, v, core)

    w = w_all.reshape(bs, HEAD, KNN)
    gidx = idx_all.reshape(bs, HEAD, KNN)
    best_indice = shuffle_index[gidx]
    phys = best_indice // VET
    exp = best_indice % VET
    vals = values[phys] * w[..., None]
    oh = jax.nn.one_hot(exp, VET, dtype=vals.dtype)
    agg = jnp.einsum('bhkd,bhke->bed', vals, oh).reshape(bs, VET * VDIM)
    out = agg @ Wv.T
    return out.reshape(b0, s0, HIDDEN)
